# Initial kernel scaffold; baseline (speedup 1.0000x reference)
import jax
import jax.numpy as jnp
from jax.experimental import pallas as pl


def kernel(lm_tokens, W1, b1, W2, b2, slots, Wg, bg, Wp, bp):
    raise NotImplementedError("write your pallas kernel here")


# speedup vs baseline: 1.8686x; 1.8686x over previous
import functools
import math

import jax
import jax.numpy as jnp
from jax import lax
from jax.experimental import pallas as pl
from jax.experimental.pallas import tpu as pltpu

NUM_SLOTS = 16
SLOT_DIM = 512
LM_HIDDEN = 2560
HIDDEN = 1024
TOP_M = 5
THR = 0.7
B = 32
P = 576
MROWS = 8

_HIGH = jax.lax.Precision.HIGHEST
_INV_SQRT2 = 1.0 / math.sqrt(2.0)


def _gelu_exact(x):
    return 0.5 * x * (1.0 + lax.erf(x * _INV_SQRT2))


def _frame_kernel(x_ref, w1_ref, b1_ref, w2c_ref, b2c_ref, wc_ref, slots_ref,
                  tc_ref, meta_ref):
    x = x_ref[0]
    h = _gelu_exact(jnp.dot(x, w1_ref[...]) + b1_ref[...])
    cand = jnp.dot(h, w2c_ref[...]) + b2c_ref[...]
    conf = jnp.dot(h, wc_ref[...])

    p_iota = lax.broadcasted_iota(jnp.int32, (P, 1), 0)
    col_iota = lax.broadcasted_iota(jnp.int32, (MROWS, P), 1)
    row_iota = lax.broadcasted_iota(jnp.int32, (MROWS, P), 0)

    sel = jnp.zeros((MROWS, P), jnp.float32)
    for m in range(TOP_M):
        mx = jnp.max(conf)
        idx_m = jnp.min(jnp.where(conf == mx, p_iota, P))
        sel = sel + jnp.where((row_iota == m) & (col_iota == idx_m), 1.0, 0.0)
        conf = jnp.where(p_iota == idx_m, -jnp.inf, conf)

    top_cand = jnp.dot(sel, cand, precision=_HIGH)
    cn = top_cand / (jnp.sqrt(jnp.sum(top_cand * top_cand, axis=1, keepdims=True)) + 1e-12)
    slots = slots_ref[...]
    sn = slots / (jnp.sqrt(jnp.sum(slots * slots, axis=1, keepdims=True)) + 1e-12)
    scores = lax.dot_general(cn, sn, (((1,), (1,)), ((), ())))

    best_score = jnp.max(scores, axis=1, keepdims=True)
    s_iota = lax.broadcasted_iota(jnp.int32, (MROWS, NUM_SLOTS), 1)
    best_idx = jnp.min(jnp.where(scores == best_score, s_iota, NUM_SLOTS),
                       axis=1, keepdims=True)

    tc_ref[0] = top_cand
    ci = lax.broadcasted_iota(jnp.int32, (MROWS, 128), 1)
    meta = (jnp.where(ci == 0, best_idx.astype(jnp.float32), 0.0)
            + jnp.where(ci == 1, best_score, 0.0))
    meta_ref[0] = meta


def _update_kernel(tc_ref, meta_ref, slots_ref, wg_ref, bg_ref, wp_ref, bp_ref,
                   out_ref):
    n = B * MROWS
    cand = tc_ref[...].reshape(n, SLOT_DIM)
    meta = meta_ref[...].reshape(n, 128)
    idx = meta[:, 0:1].astype(jnp.int32)
    score = meta[:, 1:2]
    slots = slots_ref[...]

    s_iota = lax.broadcasted_iota(jnp.int32, (n, NUM_SLOTS), 1)
    onehot = jnp.where(s_iota == idx, 1.0, 0.0)
    old = jnp.dot(onehot, slots, precision=_HIGH)

    g = jax.nn.sigmoid(
        jnp.dot(jnp.concatenate([old, cand], axis=1), wg_ref[...]) + bg_ref[...])
    upd = g * cand + (1.0 - g) * old
    vals = jnp.where(score > THR, upd, old)

    r_iota = lax.broadcasted_iota(jnp.int32, (n, 1), 0)
    bv, mv = r_iota // MROWS, r_iota % MROWS
    valid = mv < TOP_M
    rank = jnp.where(valid, 1 + bv * TOP_M + mv, -1)
    rmat = jnp.where((s_iota == idx) & valid, rank, -1)

    eye = lax.broadcasted_iota(jnp.int32, (NUM_SLOTS, NUM_SLOTS), 0) == \
        lax.broadcasted_iota(jnp.int32, (NUM_SLOTS, NUM_SLOTS), 1)
    rmat_ext = jnp.concatenate([rmat, jnp.where(eye, 0, -1)], axis=0)
    vals_ext = jnp.concatenate([vals, slots], axis=0)

    sel_rank = jnp.max(rmat_ext, axis=0, keepdims=True)
    wsel = jnp.where(rmat_ext == sel_rank, 1.0, 0.0)
    new_slots = lax.dot_general(wsel, vals_ext, (((0,), (0,)), ((), ())),
                                precision=_HIGH)

    out_ref[...] = jnp.dot(new_slots, wp_ref[...]) + bp_ref[...]


@jax.jit
def kernel(lm_tokens, W1, b1, W2, b2, slots, Wg, bg, Wp, bp):
    w2c = W2[:, :SLOT_DIM]
    wc = W2[:, SLOT_DIM:SLOT_DIM + 1]
    b2c = b2[:SLOT_DIM].reshape(1, SLOT_DIM)
    b1r = b1.reshape(1, HIDDEN)
    bgr = bg.reshape(1, SLOT_DIM)
    bpr = bp.reshape(1, LM_HIDDEN)

    top_cand, meta = pl.pallas_call(
        _frame_kernel,
        grid=(B,),
        in_specs=[
            pl.BlockSpec((1, P, LM_HIDDEN), lambda b: (b, 0, 0)),
            pl.BlockSpec((LM_HIDDEN, HIDDEN), lambda b: (0, 0)),
            pl.BlockSpec((1, HIDDEN), lambda b: (0, 0)),
            pl.BlockSpec((HIDDEN, SLOT_DIM), lambda b: (0, 0)),
            pl.BlockSpec((1, SLOT_DIM), lambda b: (0, 0)),
            pl.BlockSpec((HIDDEN, 1), lambda b: (0, 0)),
            pl.BlockSpec((NUM_SLOTS, SLOT_DIM), lambda b: (0, 0)),
        ],
        out_specs=[
            pl.BlockSpec((1, MROWS, SLOT_DIM), lambda b: (b, 0, 0)),
            pl.BlockSpec((1, MROWS, 128), lambda b: (b, 0, 0)),
        ],
        out_shape=[
            jax.ShapeDtypeStruct((B, MROWS, SLOT_DIM), jnp.float32),
            jax.ShapeDtypeStruct((B, MROWS, 128), jnp.float32),
        ],
        compiler_params=pltpu.CompilerParams(
            dimension_semantics=("arbitrary",)),
    )(lm_tokens, W1, b1r, w2c, b2c, wc, slots)

    slot_lm = pl.pallas_call(
        _update_kernel,
        out_shape=jax.ShapeDtypeStruct((NUM_SLOTS, LM_HIDDEN), jnp.float32),
    )(top_cand, meta, slots, Wg, bgr, Wp, bpr)
    return slot_lm


# speedup vs baseline: 2.7340x; 1.4632x over previous
import functools
import math

import jax
import jax.numpy as jnp
from jax import lax
from jax.experimental import pallas as pl
from jax.experimental.pallas import tpu as pltpu
from jax.experimental.pallas import tpu_sc as plsc

NUM_SLOTS = 16
SLOT_DIM = 512
LM_HIDDEN = 2560
HIDDEN = 1024
TOP_M = 5
THR = 0.7
B = 32
P = 576
MROWS = 8
LANES = 16
NSEL = B * MROWS

_HIGH = jax.lax.Precision.HIGHEST
_INV_SQRT2 = 1.0 / math.sqrt(2.0)


def _gelu_exact(x):
    return 0.5 * x * (1.0 + lax.erf(x * _INV_SQRT2))


def _conf_kernel(x_ref, w1_ref, b1_ref, wct_ref, bc_ref, conf_ref):
    x = x_ref[0]
    h = _gelu_exact(jnp.dot(x, w1_ref[...]) + b1_ref[...])
    conf = lax.dot_general(wct_ref[...], h, (((1,), (1,)), ((), ())))
    conf_ref[0] = conf + bc_ref[...]


def _sc_body(conf_hbm, x_hbm, xsel_hbm, conf_v, idx_v, bcf_v, bci_v, rows_v,
             sem):
    c = lax.axis_index("c")
    s = lax.axis_index("s")
    b = s * 2 + c
    pltpu.sync_copy(conf_hbm.at[b], conf_v)

    lanes = lax.iota(jnp.int32, 16)
    zeros = jnp.zeros((LANES,), jnp.int32)
    big = jnp.full((LANES,), 2 ** 30, jnp.int32)
    picked = []
    for m in range(TOP_M):
        def chunk_body(ci, carry, _picked=tuple(picked)):
            bv, bi = carry
            v = conf_v[pl.ds(ci * LANES, LANES)]
            g = ci * LANES + lanes
            for idx_p in _picked:
                v = jnp.where(g == idx_p, -jnp.inf, v)
            better = v > bv
            return jnp.where(better, v, bv), jnp.where(better, g, bi)

        bv, bi = lax.fori_loop(
            0, P // LANES, chunk_body,
            (jnp.full((LANES,), -jnp.inf, jnp.float32), big))
        for dstep in (8, 4, 2, 1):
            perm = (lanes + dstep) % LANES
            bcf_v[...] = bv
            bci_v[...] = bi
            pv = plsc.load_gather(bcf_v, [perm])
            pi = plsc.load_gather(bci_v, [perm])
            better = (pv > bv) | ((pv == bv) & (pi < bi))
            bv = jnp.where(better, pv, bv)
            bi = jnp.where(better, pi, bi)
        picked.append(bi)

    acc = jnp.zeros((LANES,), jnp.int32)
    for m, idx_m in enumerate(picked):
        acc = jnp.where(lanes == m, b * P + idx_m, acc)
    idx_v[...] = acc
    pltpu.async_copy(x_hbm.at[idx_v], rows_v, sem).wait()
    pltpu.sync_copy(rows_v.at[pl.ds(0, MROWS)],
                    xsel_hbm.at[pl.ds(b * MROWS, MROWS)])


def _finish_kernel(xsel_ref, w1_ref, b1_ref, w2c_ref, b2c_ref, slots_ref,
                   wg_ref, bg_ref, wp_ref, bp_ref, out_ref):
    h = _gelu_exact(jnp.dot(xsel_ref[...], w1_ref[...]) + b1_ref[...])
    cand = jnp.dot(h, w2c_ref[...]) + b2c_ref[...]

    cn = cand / (jnp.sqrt(jnp.sum(cand * cand, axis=1, keepdims=True)) + 1e-12)
    slots = slots_ref[...]
    sn = slots / (jnp.sqrt(jnp.sum(slots * slots, axis=1, keepdims=True)) + 1e-12)
    scores = lax.dot_general(cn, sn, (((1,), (1,)), ((), ())))

    best_score = jnp.max(scores, axis=1, keepdims=True)
    s_iota = lax.broadcasted_iota(jnp.int32, (NSEL, NUM_SLOTS), 1)
    idx = jnp.min(jnp.where(scores == best_score, s_iota, NUM_SLOTS),
                  axis=1, keepdims=True)

    onehot = jnp.where(s_iota == idx, 1.0, 0.0)
    old = jnp.dot(onehot, slots, precision=_HIGH)

    g = jax.nn.sigmoid(
        jnp.dot(jnp.concatenate([old, cand], axis=1), wg_ref[...]) + bg_ref[...])
    upd = g * cand + (1.0 - g) * old
    vals = jnp.where(best_score > THR, upd, old)

    r_iota = lax.broadcasted_iota(jnp.int32, (NSEL, 1), 0)
    bv, mv = r_iota // MROWS, r_iota % MROWS
    valid = mv < TOP_M
    rank = jnp.where(valid, 1 + bv * TOP_M + mv, -1)
    rmat = jnp.where((s_iota == idx) & valid, rank, -1)

    eye = lax.broadcasted_iota(jnp.int32, (NUM_SLOTS, NUM_SLOTS), 0) == \
        lax.broadcasted_iota(jnp.int32, (NUM_SLOTS, NUM_SLOTS), 1)
    rmat_ext = jnp.concatenate([rmat, jnp.where(eye, 0, -1)], axis=0)
    vals_ext = jnp.concatenate([vals, slots], axis=0)

    sel_rank = jnp.max(rmat_ext, axis=0, keepdims=True)
    wsel = jnp.where(rmat_ext == sel_rank, 1.0, 0.0)
    new_slots = lax.dot_general(wsel, vals_ext, (((0,), (0,)), ((), ())),
                                precision=_HIGH)

    out_ref[...] = jnp.dot(new_slots, wp_ref[...]) + bp_ref[...]


@jax.jit
def kernel(lm_tokens, W1, b1, W2, b2, slots, Wg, bg, Wp, bp):
    w2c = W2[:, :SLOT_DIM]
    wct = W2[:, SLOT_DIM:SLOT_DIM + 1].T
    b2c = b2[:SLOT_DIM].reshape(1, SLOT_DIM)
    bc = b2[SLOT_DIM:].reshape(1, 1)
    b1r = b1.reshape(1, HIDDEN)
    bgr = bg.reshape(1, SLOT_DIM)
    bpr = bp.reshape(1, LM_HIDDEN)

    conf = pl.pallas_call(
        _conf_kernel,
        grid=(B,),
        in_specs=[
            pl.BlockSpec((1, P, LM_HIDDEN), lambda b: (b, 0, 0)),
            pl.BlockSpec((LM_HIDDEN, HIDDEN), lambda b: (0, 0)),
            pl.BlockSpec((1, HIDDEN), lambda b: (0, 0)),
            pl.BlockSpec((1, HIDDEN), lambda b: (0, 0)),
            pl.BlockSpec((1, 1), lambda b: (0, 0)),
        ],
        out_specs=pl.BlockSpec((1, 1, P), lambda b: (b, 0, 0)),
        out_shape=jax.ShapeDtypeStruct((B, 1, P), jnp.float32),
        compiler_params=pltpu.CompilerParams(
            dimension_semantics=("arbitrary",)),
    )(lm_tokens, W1, b1r, wct, bc)

    conf2 = conf.reshape(B, P)
    x2 = lm_tokens.reshape(B * P, LM_HIDDEN)

    sc_gather = pl.kernel(
        _sc_body,
        out_type=jax.ShapeDtypeStruct((NSEL, LM_HIDDEN), jnp.float32),
        mesh=plsc.VectorSubcoreMesh(core_axis_name="c", subcore_axis_name="s",
                                    num_cores=2, num_subcores=16),
        scratch_types=[
            pltpu.VMEM((P,), jnp.float32),
            pltpu.VMEM((LANES,), jnp.int32),
            pltpu.VMEM((LANES,), jnp.float32),
            pltpu.VMEM((LANES,), jnp.int32),
            pltpu.VMEM((LANES, LM_HIDDEN), jnp.float32),
            pltpu.SemaphoreType.DMA,
        ],
        compiler_params=pltpu.CompilerParams(needs_layout_passes=False),
    )
    xsel = sc_gather(conf2, x2)

    slot_lm = pl.pallas_call(
        _finish_kernel,
        out_shape=jax.ShapeDtypeStruct((NUM_SLOTS, LM_HIDDEN), jnp.float32),
    )(xsel, W1, b1r, w2c, b2c, slots, Wg, bgr, Wp, bpr)
    return slot_lm


# speedup vs baseline: 3.0564x; 1.1179x over previous
import functools
import math

import jax
import jax.numpy as jnp
from jax import lax
from jax.experimental import pallas as pl
from jax.experimental.pallas import tpu as pltpu
from jax.experimental.pallas import tpu_sc as plsc

NUM_SLOTS = 16
SLOT_DIM = 512
LM_HIDDEN = 2560
HIDDEN = 1024
TOP_M = 5
THR = 0.7
B = 32
P = 576
MROWS = 8
LANES = 16
NSEL = B * MROWS

_HIGH = jax.lax.Precision.HIGHEST
_INV_SQRT2 = 1.0 / math.sqrt(2.0)


def _gelu_exact(x):
    return 0.5 * x * (1.0 + lax.erf(x * _INV_SQRT2))


def _conf_kernel(x_ref, w1_ref, b1_ref, wct_ref, bc_ref, conf_ref):
    x = x_ref[0]
    h = _gelu_exact(jnp.dot(x, w1_ref[...]) + b1_ref[...])
    conf = lax.dot_general(wct_ref[...], h, (((1,), (1,)), ((), ())))
    conf_ref[0] = conf + bc_ref[...]


def _combine(a, b):
    va, ga = a
    vb, gb = b
    keep_a = (va > vb) | ((va == vb) & (ga < gb))
    return jnp.where(keep_a, va, vb), jnp.where(keep_a, ga, gb)


def _sc_body(conf_hbm, x_hbm, xsel_hbm, conf_v, idx_v, bcf_v, bci_v, rows_v,
             sem):
    c = lax.axis_index("c")
    s = lax.axis_index("s")
    b = s * 2 + c
    pltpu.sync_copy(conf_hbm.at[b], conf_v)

    lanes = lax.iota(jnp.int32, 16)
    nchunks = P // LANES
    gidx = [ci * LANES + lanes for ci in range(nchunks)]
    vals = [conf_v[pl.ds(ci * LANES, LANES)] for ci in range(nchunks)]

    picked = []
    for m in range(TOP_M):
        if picked:
            last = picked[-1]
            vals = [jnp.where(g == last, -jnp.inf, v)
                    for g, v in zip(gidx, vals)]
        items = list(zip(vals, gidx))
        while len(items) > 1:
            nxt = [_combine(items[i], items[i + 1])
                   for i in range(0, len(items) - 1, 2)]
            if len(items) % 2:
                nxt.append(items[-1])
            items = nxt
        bv, bi = items[0]
        for dstep in (8, 4, 2, 1):
            perm = (lanes + dstep) % LANES
            bcf_v[...] = bv
            bci_v[...] = bi
            pv = plsc.load_gather(bcf_v, [perm])
            pi = plsc.load_gather(bci_v, [perm])
            bv, bi = _combine((bv, bi), (pv, pi))
        picked.append(bi)

    acc = jnp.zeros((LANES,), jnp.int32)
    for m, idx_m in enumerate(picked):
        acc = jnp.where(lanes == m, b * P + idx_m, acc)
    idx_v[...] = acc
    pltpu.async_copy(x_hbm.at[idx_v.at[pl.ds(0, MROWS)]], rows_v,
                     sem).wait()
    pltpu.sync_copy(rows_v, xsel_hbm.at[pl.ds(b * MROWS, MROWS)])


def _finish_kernel(xsel_ref, w1_ref, b1_ref, w2c_ref, b2c_ref, slots_ref,
                   wg_ref, bg_ref, wp_ref, bp_ref, out_ref):
    h = _gelu_exact(jnp.dot(xsel_ref[...], w1_ref[...]) + b1_ref[...])
    cand = jnp.dot(h, w2c_ref[...]) + b2c_ref[...]

    cn = cand / (jnp.sqrt(jnp.sum(cand * cand, axis=1, keepdims=True)) + 1e-12)
    slots = slots_ref[...]
    sn = slots / (jnp.sqrt(jnp.sum(slots * slots, axis=1, keepdims=True)) + 1e-12)
    scores = lax.dot_general(cn, sn, (((1,), (1,)), ((), ())))

    best_score = jnp.max(scores, axis=1, keepdims=True)
    s_iota = lax.broadcasted_iota(jnp.int32, (NSEL, NUM_SLOTS), 1)
    idx = jnp.min(jnp.where(scores == best_score, s_iota, NUM_SLOTS),
                  axis=1, keepdims=True)

    onehot = jnp.where(s_iota == idx, 1.0, 0.0)
    old = jnp.dot(onehot, slots, precision=_HIGH)

    g = jax.nn.sigmoid(
        jnp.dot(jnp.concatenate([old, cand], axis=1), wg_ref[...]) + bg_ref[...])
    upd = g * cand + (1.0 - g) * old
    vals = jnp.where(best_score > THR, upd, old)

    r_iota = lax.broadcasted_iota(jnp.int32, (NSEL, 1), 0)
    bv, mv = r_iota // MROWS, r_iota % MROWS
    valid = mv < TOP_M
    rank = jnp.where(valid, 1 + bv * TOP_M + mv, -1)
    rmat = jnp.where((s_iota == idx) & valid, rank, -1)

    eye = lax.broadcasted_iota(jnp.int32, (NUM_SLOTS, NUM_SLOTS), 0) == \
        lax.broadcasted_iota(jnp.int32, (NUM_SLOTS, NUM_SLOTS), 1)
    rmat_ext = jnp.concatenate([rmat, jnp.where(eye, 0, -1)], axis=0)
    vals_ext = jnp.concatenate([vals, slots], axis=0)

    sel_rank = jnp.max(rmat_ext, axis=0, keepdims=True)
    wsel = jnp.where(rmat_ext == sel_rank, 1.0, 0.0)
    new_slots = lax.dot_general(wsel, vals_ext, (((0,), (0,)), ((), ())),
                                precision=_HIGH)

    out_ref[...] = jnp.dot(new_slots, wp_ref[...]) + bp_ref[...]


@jax.jit
def kernel(lm_tokens, W1, b1, W2, b2, slots, Wg, bg, Wp, bp):
    w2c = W2[:, :SLOT_DIM]
    wct = W2[:, SLOT_DIM:SLOT_DIM + 1].T
    b2c = b2[:SLOT_DIM].reshape(1, SLOT_DIM)
    bc = b2[SLOT_DIM:].reshape(1, 1)
    b1r = b1.reshape(1, HIDDEN)
    bgr = bg.reshape(1, SLOT_DIM)
    bpr = bp.reshape(1, LM_HIDDEN)

    conf = pl.pallas_call(
        _conf_kernel,
        grid=(B,),
        in_specs=[
            pl.BlockSpec((1, P, LM_HIDDEN), lambda b: (b, 0, 0)),
            pl.BlockSpec((LM_HIDDEN, HIDDEN), lambda b: (0, 0)),
            pl.BlockSpec((1, HIDDEN), lambda b: (0, 0)),
            pl.BlockSpec((1, HIDDEN), lambda b: (0, 0)),
            pl.BlockSpec((1, 1), lambda b: (0, 0)),
        ],
        out_specs=pl.BlockSpec((1, 1, P), lambda b: (b, 0, 0)),
        out_shape=jax.ShapeDtypeStruct((B, 1, P), jnp.float32),
        compiler_params=pltpu.CompilerParams(
            dimension_semantics=("arbitrary",)),
    )(lm_tokens, W1, b1r, wct, bc)

    conf2 = conf.reshape(B, P)
    x2 = lm_tokens.reshape(B * P, LM_HIDDEN)

    sc_gather = pl.kernel(
        _sc_body,
        out_type=jax.ShapeDtypeStruct((NSEL, LM_HIDDEN), jnp.float32),
        mesh=plsc.VectorSubcoreMesh(core_axis_name="c", subcore_axis_name="s",
                                    num_cores=2, num_subcores=16),
        scratch_types=[
            pltpu.VMEM((P,), jnp.float32),
            pltpu.VMEM((LANES,), jnp.int32),
            pltpu.VMEM((LANES,), jnp.float32),
            pltpu.VMEM((LANES,), jnp.int32),
            pltpu.VMEM((MROWS, LM_HIDDEN), jnp.float32),
            pltpu.SemaphoreType.DMA,
        ],
        compiler_params=pltpu.CompilerParams(needs_layout_passes=False),
    )
    xsel = sc_gather(conf2, x2)

    slot_lm = pl.pallas_call(
        _finish_kernel,
        out_shape=jax.ShapeDtypeStruct((NUM_SLOTS, LM_HIDDEN), jnp.float32),
    )(xsel, W1, b1r, w2c, b2c, slots, Wg, bgr, Wp, bpr)
    return slot_lm
